# baseline (speedup 1.0000x reference)
import functools

import jax
import jax.numpy as jnp
from jax.experimental import pallas as pl
from jax.experimental.pallas import tpu as pltpu
from jax.experimental.pallas import tpu_sc as plsc

B, H, D = 64, 32, 64
NUM_SLOTS = 128
NG = 3
NS = 3
NW = 32
JMAX = 4


QROWS = NUM_SLOTS * 4
QW = 32768
NROUND = (NUM_SLOTS - B) * 4 // NW // 2


def _sc_copy_body(cache4_hbm, comp3_hbm, out4_hbm, idx_v, stage):
    wid = jax.lax.axis_index("s") * 2 + jax.lax.axis_index("c")
    pltpu.sync_copy(comp3_hbm.at[wid], idx_v)
    for j in range(NROUND):
        pltpu.sync_copy(cache4_hbm.at[idx_v.at[j]], stage)
        pltpu.sync_copy(stage, out4_hbm.at[idx_v.at[j]])


@functools.lru_cache(maxsize=1)
def _make_sc_copy():
    return functools.partial(
        pl.kernel,
        out_type=jax.ShapeDtypeStruct((QROWS, QW), jnp.float32),
        mesh=plsc.VectorSubcoreMesh(core_axis_name="c",
                                    subcore_axis_name="s"),
        scratch_types=[
            pltpu.VMEM((NROUND, 2), jnp.int32),
            pltpu.VMEM((2, QW), jnp.float32),
        ],
    )(_sc_copy_body)


def _tc_body(slot_ref, cache_ref, partial_hbm, q_ref, k_ref, v_ref,
             slope_ref, newc_ref, outq_ref):
    b = pl.program_id(0)
    ratio = jnp.exp(-slope_ref[0])
    kv_old = cache_ref[0]
    k3 = k_ref[b, :, 0, :]
    v3 = v_ref[b, :, 0, :]
    q3 = q_ref[b, :, 0, :]
    kv_new = (k3[:, :, None] * v3[:, None, :]
              + ratio[:, None, None] * kv_old)
    newc_ref[0] = kv_new
    outq_ref[b, :, 0, :] = jnp.sum(q3[:, :, None] * kv_new, axis=1)


def kernel(q, k, v, kv_caches, slope_rate, slot_idx):
    slot_idx = slot_idx.astype(jnp.int32)
    touched = jnp.zeros((NUM_SLOTS,), jnp.int32).at[slot_idx].set(1)
    perm = jnp.argsort(touched, stable=True).astype(jnp.int32)
    unt = perm[:NUM_SLOTS - B]
    qs = (unt[:, None] * 4
          + jnp.arange(4, dtype=jnp.int32)[None, :]).reshape(-1)
    comp3 = qs.reshape(NW, NROUND, 2)

    cache4 = kv_caches.reshape(QROWS, QW)
    partial = _make_sc_copy()(cache4, comp3).reshape(
        NUM_SLOTS, H, D, D)

    slope2 = slope_rate.reshape(1, H)
    grid_spec = pltpu.PrefetchScalarGridSpec(
        num_scalar_prefetch=1,
        grid=(B,),
        in_specs=[
            pl.BlockSpec((1, H, D, D),
                         lambda b, slots: (slots[b], 0, 0, 0)),
            pl.BlockSpec(memory_space=pltpu.MemorySpace.HBM),
            pl.BlockSpec((B, H, 1, D), lambda b, slots: (0, 0, 0, 0)),
            pl.BlockSpec((B, H, 1, D), lambda b, slots: (0, 0, 0, 0)),
            pl.BlockSpec((B, H, 1, D), lambda b, slots: (0, 0, 0, 0)),
            pl.BlockSpec((1, H), lambda b, slots: (0, 0)),
        ],
        out_specs=[
            pl.BlockSpec((1, H, D, D),
                         lambda b, slots: (slots[b], 0, 0, 0)),
            pl.BlockSpec((B, H, 1, D), lambda b, slots: (0, 0, 0, 0)),
        ],
    )
    new_cache, output = pl.pallas_call(
        _tc_body,
        grid_spec=grid_spec,
        out_shape=[
            jax.ShapeDtypeStruct((NUM_SLOTS, H, D, D), jnp.float32),
            jax.ShapeDtypeStruct((B, H, 1, D), jnp.float32),
        ],
        input_output_aliases={2: 0},
    )(slot_idx, kv_caches, partial, q, k, v, slope2)
    return output, new_cache


# speedup vs baseline: 1.6945x; 1.6945x over previous
import jax
import jax.numpy as jnp
from jax.experimental import pallas as pl
from jax.experimental.pallas import tpu as pltpu

B, H, D = 64, 32, 64
NUM_SLOTS = 128
BS = 8


def _slot_kernel(inv_ref, cache_ref, q_ref, k_ref, v_ref, slope_ref,
                 newc_ref, out_ref):
    s = pl.program_id(0)
    ratio = jnp.exp(-slope_ref[0])
    kv_old = cache_ref[...]

    for j in range(BS):
        b = inv_ref[BS * s + j]
        kvo = kv_old[j]

        @pl.when(b >= 0)
        def _update(b=b, kvo=kvo, j=j):
            k3 = k_ref[b, :, 0, :]
            v3 = v_ref[b, :, 0, :]
            q3 = q_ref[b, :, 0, :]
            kv_new = (k3[:, :, None] * v3[:, None, :]
                      + ratio[:, None, None] * kvo)
            newc_ref[j] = kv_new
            out_ref[b, :, 0, :] = jnp.sum(q3[:, :, None] * kv_new, axis=1)

        @pl.when(b < 0)
        def _copy(kvo=kvo, j=j):
            newc_ref[j] = kvo


def kernel(q, k, v, kv_caches, slope_rate, slot_idx):
    slot_idx = slot_idx.astype(jnp.int32)
    inv = jnp.full((NUM_SLOTS,), -1, jnp.int32).at[slot_idx].set(
        jnp.arange(B, dtype=jnp.int32))
    slope2 = slope_rate.reshape(1, H)

    grid_spec = pltpu.PrefetchScalarGridSpec(
        num_scalar_prefetch=1,
        grid=(NUM_SLOTS // BS,),
        in_specs=[
            pl.BlockSpec((BS, H, D, D), lambda s, inv: (s, 0, 0, 0)),
            pl.BlockSpec((B, H, 1, D), lambda s, inv: (0, 0, 0, 0)),
            pl.BlockSpec((B, H, 1, D), lambda s, inv: (0, 0, 0, 0)),
            pl.BlockSpec((B, H, 1, D), lambda s, inv: (0, 0, 0, 0)),
            pl.BlockSpec((1, H), lambda s, inv: (0, 0)),
        ],
        out_specs=[
            pl.BlockSpec((BS, H, D, D), lambda s, inv: (s, 0, 0, 0)),
            pl.BlockSpec((B, H, 1, D), lambda s, inv: (0, 0, 0, 0)),
        ],
    )
    new_cache, output = pl.pallas_call(
        _slot_kernel,
        grid_spec=grid_spec,
        out_shape=[
            jax.ShapeDtypeStruct((NUM_SLOTS, H, D, D), jnp.float32),
            jax.ShapeDtypeStruct((B, H, 1, D), jnp.float32),
        ],
    )(inv, kv_caches, q, k, v, slope2)
    return output, new_cache
